# baseline (speedup 1.0000x reference)
import dataclasses
import functools

import jax
import jax.numpy as jnp
from jax import lax
from jax.experimental import pallas as pl
from jax.experimental.pallas import tpu as pltpu
from jax.experimental.pallas import tpu_sc as plsc

_N = 20000
_NV = _N // 16
_P = 8192
_PPAD = _P + 32
_K = 2000
_KV = _K // 16
_BINS = 2048
_BV = _BINS // 16
_INT_MAX = 2**31 - 1


def _sc_body(starts_hbm, ends_hbm, bits_hbm, nts_hbm,
             sel_hbm, outs_hbm, oute_hbm, outsc_hbm,
             akey, aidx, bkey, bidx, sev, hist, tab,
             selv, ost, oen, osc, ntsv):
    cid = lax.axis_index("c")
    sid = lax.axis_index("s")

    @pl.when((cid == 0) & (sid == 0))
    def _main():
        iota = lax.iota(jnp.int32, 16)
        cnt0, _ = plsc.scan_count(jnp.zeros((16,), jnp.int32))
        cal = jnp.min(cnt0)

        pltpu.sync_copy(bits_hbm, akey)
        pltpu.sync_copy(starts_hbm, bkey)
        pltpu.sync_copy(ends_hbm, bidx.at[pl.ds(0, _N)])
        pltpu.sync_copy(nts_hbm, ntsv)
        nts = jnp.minimum(ntsv[pl.ds(0, 16)][0], jnp.int32(_K))
        lane0 = iota == 0

        @pl.loop(0, _NV)
        def _prep(v):
            sl = pl.ds(v * 16, 16)
            u = akey[sl]
            akey[sl] = jnp.where(u >= 0, jnp.int32(_INT_MAX) - u, u)
            aidx[sl] = v * 16 + iota
            s = bkey[sl]
            e = bidx[sl]
            sev[sl] = s | ((e - s) << 13)

        @pl.loop(0, _PPAD // 16)
        def _init_tables(v):
            tab[pl.ds(v * 16, 16)] = jnp.full((16,), _P << 14, jnp.int32)

        def radix_pass(skey, sidx, dkey, didx, shift, mask):
            @pl.loop(0, _BV)
            def _clr(v):
                hist[pl.ds(v * 16, 16)] = jnp.zeros((16,), jnp.int32)

            @pl.loop(0, _NV)
            def _count(v):
                sl = pl.ds(v * 16, 16)
                d = lax.shift_right_logical(skey[sl], shift) & mask
                cnt, last = plsc.scan_count(d)
                base = plsc.load_gather(hist, [d])
                plsc.store_scatter(hist, [d], base + (cnt - cal) + 1, mask=last)

            def _scan(v, carry):
                sl = pl.ds(v * 16, 16)
                h = hist[sl]
                inc = plsc.cumsum(h)
                hist[sl] = inc - h + carry
                return carry + jnp.max(inc)

            lax.fori_loop(0, _BV, _scan, jnp.int32(0))

            @pl.loop(0, _NV)
            def _place(v):
                sl = pl.ds(v * 16, 16)
                k = skey[sl]
                ix = sidx[sl]
                d = lax.shift_right_logical(k, shift) & mask
                cnt, last = plsc.scan_count(d)
                base = plsc.load_gather(hist, [d])
                pos = base + (cnt - cal)
                plsc.store_scatter(dkey, [pos], k)
                plsc.store_scatter(didx, [pos], ix)
                plsc.store_scatter(hist, [d], base + (cnt - cal) + 1, mask=last)

        radix_pass(akey, aidx, bkey, bidx, 0, 2047)
        radix_pass(bkey, bidx, akey, aidx, 11, 2047)
        radix_pass(akey, aidx, bkey, bidx, 22, 1023)

        @pl.loop(0, _NV)
        def _clr_flags(v):
            akey[pl.ds(v * 16, 16)] = jnp.zeros((16,), jnp.int32)

        mask_gt0 = iota > 0

        def greedy_cond(st):
            i, count, ind, se = st
            return (i < _N) & (count < nts)

        def greedy_body(st):
            i, count, ind, se = st
            nind = bidx[pl.ds(i + 1, 16)][0]
            nse = sev[pl.ds(nind, 16)][0]
            cs = se & jnp.int32(_P - 1)
            ce = cs + lax.shift_right_logical(se, 13)
            j0 = cs + iota
            j1 = j0 + 16
            t0 = tab[pl.ds(cs, 16)]
            t1 = tab[pl.ds(cs + 16, 16)]
            s0 = t0 & jnp.int32(0x3FFF)
            s1 = t1 & jnp.int32(0x3FFF)
            e0 = lax.shift_right_logical(t0, 14)
            e1 = lax.shift_right_logical(t1, 14)
            c0 = (j0 <= ce) & ((mask_gt0 & (s0 > ce)) | ((j0 < ce) & (e0 < cs)))
            c1 = (j1 <= ce) & ((s1 > ce) | ((j1 < ce) & (e1 < cs)))
            pop = plsc.all_reduce_population_count(c0 | c1)[0]
            take = pop == 0

            @pl.when(take)
            def _():
                indv = jnp.broadcast_to(ind, (16,))
                csv = jnp.broadcast_to(cs, (16,))
                cev = jnp.broadcast_to(ce, (16,))
                plsc.store_scatter(akey, [indv], jnp.full((16,), 1, jnp.int32),
                                   mask=lane0)
                old0 = t0[0]
                new0 = (jnp.maximum(old0 & jnp.int32(0x3FFF), ce)
                        | ((lax.shift_right_logical(old0, 14)) << 14))
                plsc.store_scatter(tab, [csv], jnp.broadcast_to(new0, (16,)),
                                   mask=lane0)
                oldc = plsc.load_gather(tab, [cev])[0]
                newc = ((oldc & jnp.int32(0x3FFF))
                        | (jnp.minimum(lax.shift_right_logical(oldc, 14), cs)
                           << 14))
                plsc.store_scatter(tab, [cev], jnp.broadcast_to(newc, (16,)),
                                   mask=lane0)

            return i + 1, count + take.astype(jnp.int32), nind, nse

        ind0 = bidx[pl.ds(0, 16)][0]
        se0 = sev[pl.ds(ind0, 16)][0]
        _, count, _, _ = lax.while_loop(
            greedy_cond, greedy_body, (jnp.int32(0), jnp.int32(0), ind0, se0))

        @pl.loop(0, _KV)
        def _sel_init(v):
            selv[pl.ds(v * 16, 16)] = jnp.full((16,), _INT_MAX, jnp.int32)

        def comp_body(v, off):
            sl = pl.ds(v * 16, 16)
            m = akey[sl] > 0
            c = plsc.cumsum(m.astype(jnp.int32))
            plsc.store_scatter(selv, [off + c - 1], v * 16 + iota, mask=m)
            return off + jnp.max(c)

        lax.fori_loop(0, _NV, comp_body, jnp.int32(0))
        first = selv[pl.ds(0, 16)][0]

        @pl.loop(0, _KV)
        def _fill(v):
            sl = pl.ds(v * 16, 16)
            lanes = v * 16 + iota
            cur = selv[sl]
            selv[sl] = jnp.where(lanes < count, cur, first)

        pltpu.sync_copy(bits_hbm, bkey)

        @pl.loop(0, _KV)
        def _gather_out(v):
            sl = pl.ds(v * 16, 16)
            sv = selv[sl]
            se = plsc.load_gather(sev, [sv])
            cs = se & jnp.int32(_P - 1)
            ost[sl] = cs
            oen[sl] = cs + lax.shift_right_logical(se, 13)
            osc[sl] = plsc.load_gather(bkey, [sv])

        pltpu.sync_copy(selv, sel_hbm)
        pltpu.sync_copy(ost, outs_hbm)
        pltpu.sync_copy(oen, oute_hbm)
        pltpu.sync_copy(osc, outsc_hbm)


_cp = pltpu.CompilerParams()
if "needs_layout_passes" in pltpu.CompilerParams.__dataclass_fields__:
    _cp = dataclasses.replace(_cp, needs_layout_passes=False)

_decode = functools.partial(
    pl.kernel,
    compiler_params=_cp,
    out_type=(
        jax.ShapeDtypeStruct((_K,), jnp.int32),
        jax.ShapeDtypeStruct((_K,), jnp.int32),
        jax.ShapeDtypeStruct((_K,), jnp.int32),
        jax.ShapeDtypeStruct((_K,), jnp.int32),
    ),
    mesh=plsc.VectorSubcoreMesh(core_axis_name="c", subcore_axis_name="s"),
    scratch_types=[
        pltpu.VMEM((_N,), jnp.int32),
        pltpu.VMEM((_N,), jnp.int32),
        pltpu.VMEM((_N,), jnp.int32),
        pltpu.VMEM((_N + 16,), jnp.int32),
        pltpu.VMEM((_N + 16,), jnp.int32),
        pltpu.VMEM((_BINS,), jnp.int32),
        pltpu.VMEM((_PPAD,), jnp.int32),
        pltpu.VMEM((_K,), jnp.int32),
        pltpu.VMEM((_K,), jnp.int32),
        pltpu.VMEM((_K,), jnp.int32),
        pltpu.VMEM((_K,), jnp.int32),
        pltpu.VMEM((16,), jnp.int32),
    ],
)(_sc_body)


def kernel(candidate_starts, candidate_ends, candidate_mention_scores,
           num_top_spans):
    bits = lax.bitcast_convert_type(candidate_mention_scores, jnp.int32)
    nts = jnp.broadcast_to(
        jnp.asarray(num_top_spans, jnp.int32).reshape(()), (16,))
    sel, ts, te, tb = _decode(candidate_starts, candidate_ends, bits, nts)
    return sel, ts, te, lax.bitcast_convert_type(tb, jnp.float32)


# speedup vs baseline: 1.4891x; 1.4891x over previous
import dataclasses
import functools

import jax
import jax.numpy as jnp
from jax import lax
from jax.experimental import pallas as pl
from jax.experimental.pallas import tpu as pltpu
from jax.experimental.pallas import tpu_sc as plsc

_N = 20000
_NV = _N // 16
_P = 8192
_PPAD = _P + 32
_K = 2000
_KV = _K // 16
_BINS = 2048
_BV = _BINS // 16
_INT_MAX = 2**31 - 1


def _sc_body(starts_hbm, ends_hbm, bits_hbm, nts_hbm,
             sel_hbm, outs_hbm, oute_hbm, outsc_hbm,
             akey, aidx, bkey, bidx, sev, hist, tab,
             selv, ost, oen, osc, ntsv, csbuf, cebuf):
    cid = lax.axis_index("c")
    sid = lax.axis_index("s")

    @pl.when((cid == 0) & (sid == 0))
    def _main():
        iota = lax.iota(jnp.int32, 16)
        cnt0, _ = plsc.scan_count(jnp.zeros((16,), jnp.int32))
        cal = jnp.min(cnt0)

        pltpu.sync_copy(bits_hbm, akey)
        pltpu.sync_copy(starts_hbm, bkey)
        pltpu.sync_copy(ends_hbm, bidx.at[pl.ds(0, _N)])
        pltpu.sync_copy(nts_hbm, ntsv)
        nts = jnp.minimum(ntsv[pl.ds(0, 16)][0], jnp.int32(_K))
        lane0 = iota == 0

        @pl.loop(0, _NV)
        def _prep(v):
            sl = pl.ds(v * 16, 16)
            u = akey[sl]
            akey[sl] = jnp.where(u >= 0, jnp.int32(_INT_MAX) - u, u)
            aidx[sl] = v * 16 + iota
            s = bkey[sl]
            e = bidx[sl]
            sev[sl] = s | ((e - s) << 13)

        @pl.loop(0, _PPAD // 16)
        def _init_tables(v):
            tab[pl.ds(v * 16, 16)] = jnp.full((16,), _P << 14, jnp.int32)

        def radix_pass(skey, sidx, dkey, didx, shift, mask):
            @pl.loop(0, _BV)
            def _clr(v):
                hist[pl.ds(v * 16, 16)] = jnp.zeros((16,), jnp.int32)

            @pl.loop(0, _NV)
            def _count(v):
                sl = pl.ds(v * 16, 16)
                d = lax.shift_right_logical(skey[sl], shift) & mask
                cnt, last = plsc.scan_count(d)
                base = plsc.load_gather(hist, [d])
                plsc.store_scatter(hist, [d], base + (cnt - cal) + 1, mask=last)

            def _scan(v, carry):
                sl = pl.ds(v * 16, 16)
                h = hist[sl]
                inc = plsc.cumsum(h)
                hist[sl] = inc - h + carry
                return carry + jnp.max(inc)

            lax.fori_loop(0, _BV, _scan, jnp.int32(0))

            @pl.loop(0, _NV)
            def _place(v):
                sl = pl.ds(v * 16, 16)
                k = skey[sl]
                ix = sidx[sl]
                d = lax.shift_right_logical(k, shift) & mask
                cnt, last = plsc.scan_count(d)
                base = plsc.load_gather(hist, [d])
                pos = base + (cnt - cal)
                plsc.store_scatter(dkey, [pos], k)
                plsc.store_scatter(didx, [pos], ix)
                plsc.store_scatter(hist, [d], base + (cnt - cal) + 1, mask=last)

        radix_pass(akey, aidx, bkey, bidx, 0, 2047)
        radix_pass(bkey, bidx, akey, aidx, 11, 2047)
        radix_pass(akey, aidx, bkey, bidx, 22, 1023)

        @pl.loop(0, _NV)
        def _clr_flags(v):
            akey[pl.ds(v * 16, 16)] = jnp.zeros((16,), jnp.int32)

        mask_gt0 = iota > 0
        iota16 = iota + 16
        ones_v = jnp.full((16,), 1, jnp.int32)
        ntsb = jnp.broadcast_to(nts, (16,))
        low14 = jnp.int32(0x3FFF)

        def greedy_cond(st):
            i, countv = st
            return (i < _N) & (countv[0] < nts)

        def greedy_body(st):
            i, countv = st
            ind_vec = bidx[pl.ds(i, 16)]
            se_vec = plsc.load_gather(sev, [ind_vec])
            csbuf[pl.ds(16, 16)] = se_vec & jnp.int32(_P - 1)
            cebuf[pl.ds(16, 16)] = ((se_vec & jnp.int32(_P - 1))
                                    + lax.shift_right_logical(se_vec, 13))
            takebits = jnp.zeros((16,), jnp.bool_)
            for k in range(16):
                kvec = jnp.full((16,), 16 + k, jnp.int32)
                csb = plsc.load_gather(csbuf, [kvec])
                ceb = plsc.load_gather(cebuf, [kvec])
                j0 = csb + iota
                j1 = csb + iota16
                t0 = plsc.load_gather(tab, [j0])
                t1 = plsc.load_gather(tab, [j1])
                s0 = t0 & low14
                s1 = t1 & low14
                e0 = lax.shift_right_logical(t0, 14)
                e1 = lax.shift_right_logical(t1, 14)
                c0 = (j0 <= ceb) & ((mask_gt0 & (s0 > ceb))
                                    | ((j0 < ceb) & (e0 < csb)))
                c1 = (j1 <= ceb) & ((s1 > ceb) | ((j1 < ceb) & (e1 < csb)))
                ncross = plsc.all_reduce_population_count(c0 | c1)
                take = (ncross == 0) & (countv < ntsb)
                tm = take & lane0
                takebits = takebits | (take & (iota == k))
                new0 = (jnp.maximum(t0 & low14, ceb)
                        | (lax.shift_right_logical(t0, 14) << 14))
                plsc.store_scatter(tab, [j0], new0, mask=tm)
                tc = plsc.load_gather(tab, [ceb])
                newc = ((tc & low14)
                        | (jnp.minimum(lax.shift_right_logical(tc, 14), csb)
                           << 14))
                plsc.store_scatter(tab, [ceb], newc, mask=tm)
                countv = countv + take.astype(jnp.int32)
            plsc.store_scatter(akey, [ind_vec], ones_v, mask=takebits)
            return i + 16, countv

        _, countv = lax.while_loop(
            greedy_cond, greedy_body,
            (jnp.int32(0), jnp.zeros((16,), jnp.int32)))
        count = countv[0]

        @pl.loop(0, _KV)
        def _sel_init(v):
            selv[pl.ds(v * 16, 16)] = jnp.full((16,), _INT_MAX, jnp.int32)

        def comp_body(v, off):
            sl = pl.ds(v * 16, 16)
            m = akey[sl] > 0
            c = plsc.cumsum(m.astype(jnp.int32))
            plsc.store_scatter(selv, [off + c - 1], v * 16 + iota, mask=m)
            return off + jnp.max(c)

        lax.fori_loop(0, _NV, comp_body, jnp.int32(0))
        first = selv[pl.ds(0, 16)][0]

        @pl.loop(0, _KV)
        def _fill(v):
            sl = pl.ds(v * 16, 16)
            lanes = v * 16 + iota
            cur = selv[sl]
            selv[sl] = jnp.where(lanes < count, cur, first)

        pltpu.sync_copy(bits_hbm, bkey)

        @pl.loop(0, _KV)
        def _gather_out(v):
            sl = pl.ds(v * 16, 16)
            sv = selv[sl]
            se = plsc.load_gather(sev, [sv])
            cs = se & jnp.int32(_P - 1)
            ost[sl] = cs
            oen[sl] = cs + lax.shift_right_logical(se, 13)
            osc[sl] = plsc.load_gather(bkey, [sv])

        pltpu.sync_copy(selv, sel_hbm)
        pltpu.sync_copy(ost, outs_hbm)
        pltpu.sync_copy(oen, oute_hbm)
        pltpu.sync_copy(osc, outsc_hbm)


_cp = pltpu.CompilerParams()
if "needs_layout_passes" in pltpu.CompilerParams.__dataclass_fields__:
    _cp = dataclasses.replace(_cp, needs_layout_passes=False)

_decode = functools.partial(
    pl.kernel,
    compiler_params=_cp,
    out_type=(
        jax.ShapeDtypeStruct((_K,), jnp.int32),
        jax.ShapeDtypeStruct((_K,), jnp.int32),
        jax.ShapeDtypeStruct((_K,), jnp.int32),
        jax.ShapeDtypeStruct((_K,), jnp.int32),
    ),
    mesh=plsc.VectorSubcoreMesh(core_axis_name="c", subcore_axis_name="s"),
    scratch_types=[
        pltpu.VMEM((_N,), jnp.int32),
        pltpu.VMEM((_N,), jnp.int32),
        pltpu.VMEM((_N,), jnp.int32),
        pltpu.VMEM((_N + 16,), jnp.int32),
        pltpu.VMEM((_N + 16,), jnp.int32),
        pltpu.VMEM((_BINS,), jnp.int32),
        pltpu.VMEM((_PPAD,), jnp.int32),
        pltpu.VMEM((_K,), jnp.int32),
        pltpu.VMEM((_K,), jnp.int32),
        pltpu.VMEM((_K,), jnp.int32),
        pltpu.VMEM((_K,), jnp.int32),
        pltpu.VMEM((16,), jnp.int32),
        pltpu.VMEM((32,), jnp.int32),
        pltpu.VMEM((32,), jnp.int32),
    ],
)(_sc_body)


def kernel(candidate_starts, candidate_ends, candidate_mention_scores,
           num_top_spans):
    bits = lax.bitcast_convert_type(candidate_mention_scores, jnp.int32)
    nts = jnp.broadcast_to(
        jnp.asarray(num_top_spans, jnp.int32).reshape(()), (16,))
    sel, ts, te, tb = _decode(candidate_starts, candidate_ends, bits, nts)
    return sel, ts, te, lax.bitcast_convert_type(tb, jnp.float32)


# speedup vs baseline: 1.5621x; 1.0490x over previous
import dataclasses
import functools

import jax
import jax.numpy as jnp
from jax import lax
from jax.experimental import pallas as pl
from jax.experimental.pallas import tpu as pltpu
from jax.experimental.pallas import tpu_sc as plsc

_N = 20000
_NV = _N // 16
_P = 8192
_PPAD = _P + 32
_K = 2000
_KV = _K // 16
_BINS = 2048
_BV = _BINS // 16
_INT_MAX = 2**31 - 1


def _sc_body(starts_hbm, ends_hbm, bits_hbm, nts_hbm,
             sel_hbm, outs_hbm, oute_hbm, outsc_hbm,
             akey, aidx, bkey, bidx, sev, hist, tab,
             selv, ost, oen, osc, ntsv, csbuf, cebuf):
    cid = lax.axis_index("c")
    sid = lax.axis_index("s")

    @pl.when((cid == 0) & (sid == 0))
    def _main():
        iota = lax.iota(jnp.int32, 16)
        cnt0, _ = plsc.scan_count(jnp.zeros((16,), jnp.int32))
        cal = jnp.min(cnt0)

        pltpu.sync_copy(bits_hbm, akey)
        pltpu.sync_copy(starts_hbm, bkey)
        pltpu.sync_copy(ends_hbm, bidx.at[pl.ds(0, _N)])
        pltpu.sync_copy(nts_hbm, ntsv)
        nts = jnp.minimum(ntsv[pl.ds(0, 16)][0], jnp.int32(_K))
        lane0 = iota == 0

        @pl.loop(0, _PPAD // 16)
        def _init_tables(v):
            tab[pl.ds(v * 16, 16)] = jnp.full((16,), _P << 14, jnp.int32)

        def radix_pass(skey, sidx, dkey, didx, shift, mask,
                       fuse_prep=False, clear_src=False):
            @pl.loop(0, _BV)
            def _clr(v):
                hist[pl.ds(v * 16, 16)] = jnp.zeros((16,), jnp.int32)

            @pl.loop(0, _NV)
            def _count(v):
                sl = pl.ds(v * 16, 16)
                if fuse_prep:
                    u = skey[sl]
                    key = jnp.where(u >= 0, jnp.int32(_INT_MAX) - u, u)
                    skey[sl] = key
                    sidx[sl] = v * 16 + iota
                    s = bkey[sl]
                    e = bidx[sl]
                    sev[sl] = s | ((e - s) << 13)
                else:
                    key = skey[sl]
                d = lax.shift_right_logical(key, shift) & mask
                cnt, last = plsc.scan_count(d)
                base = plsc.load_gather(hist, [d])
                plsc.store_scatter(hist, [d], base + (cnt - cal) + 1, mask=last)

            def _scan(v, carry):
                sl = pl.ds(v * 16, 16)
                h = hist[sl]
                inc = plsc.cumsum(h)
                hist[sl] = inc - h + carry
                return carry + jnp.max(inc)

            lax.fori_loop(0, _BV, _scan, jnp.int32(0))

            @pl.loop(0, _NV)
            def _place(v):
                sl = pl.ds(v * 16, 16)
                k = skey[sl]
                ix = sidx[sl]
                if clear_src:
                    skey[sl] = jnp.zeros((16,), jnp.int32)
                d = lax.shift_right_logical(k, shift) & mask
                cnt, last = plsc.scan_count(d)
                base = plsc.load_gather(hist, [d])
                pos = base + (cnt - cal)
                plsc.store_scatter(dkey, [pos], k)
                plsc.store_scatter(didx, [pos], ix)
                plsc.store_scatter(hist, [d], base + (cnt - cal) + 1, mask=last)

        radix_pass(akey, aidx, bkey, bidx, 0, 2047, fuse_prep=True)
        radix_pass(bkey, bidx, akey, aidx, 11, 2047)
        radix_pass(akey, aidx, bkey, bidx, 22, 1023, clear_src=True)

        mask_gt0 = iota > 0
        iota16 = iota + 16
        ones_v = jnp.full((16,), 1, jnp.int32)
        ntsb = jnp.broadcast_to(nts, (16,))
        low14 = jnp.int32(0x3FFF)

        def greedy_cond(st):
            i, countv = st
            return (i < _N) & (countv[0] < nts)

        def greedy_body(st):
            i, countv = st
            ind_vec = bidx[pl.ds(i, 16)]
            se_vec = plsc.load_gather(sev, [ind_vec])
            csbuf[pl.ds(16, 16)] = se_vec & jnp.int32(_P - 1)
            cebuf[pl.ds(16, 16)] = ((se_vec & jnp.int32(_P - 1))
                                    + lax.shift_right_logical(se_vec, 13))
            takebits = jnp.zeros((16,), jnp.bool_)
            for k in range(16):
                kvec = jnp.full((16,), 16 + k, jnp.int32)
                csb = plsc.load_gather(csbuf, [kvec])
                ceb = plsc.load_gather(cebuf, [kvec])
                j0 = csb + iota
                j1 = csb + iota16
                t0 = plsc.load_gather(tab, [j0])
                t1 = plsc.load_gather(tab, [j1])
                s0 = t0 & low14
                s1 = t1 & low14
                e0 = lax.shift_right_logical(t0, 14)
                e1 = lax.shift_right_logical(t1, 14)
                c0 = (j0 <= ceb) & ((mask_gt0 & (s0 > ceb))
                                    | ((j0 < ceb) & (e0 < csb)))
                c1 = (j1 <= ceb) & ((s1 > ceb) | ((j1 < ceb) & (e1 < csb)))
                ncross = plsc.all_reduce_population_count(c0 | c1)
                take = (ncross == 0) & (countv < ntsb)
                tm = take & lane0
                takebits = takebits | (take & (iota == k))
                new0 = (jnp.maximum(t0 & low14, ceb)
                        | (lax.shift_right_logical(t0, 14) << 14))
                plsc.store_scatter(tab, [j0], new0, mask=tm)
                tc = plsc.load_gather(tab, [ceb])
                newc = ((tc & low14)
                        | (jnp.minimum(lax.shift_right_logical(tc, 14), csb)
                           << 14))
                plsc.store_scatter(tab, [ceb], newc, mask=tm)
                countv = countv + take.astype(jnp.int32)
            plsc.store_scatter(akey, [ind_vec], ones_v, mask=takebits)
            return i + 16, countv

        _, countv = lax.while_loop(
            greedy_cond, greedy_body,
            (jnp.int32(0), jnp.zeros((16,), jnp.int32)))
        count = countv[0]

        def comp_body(v, offv):
            sl = pl.ds(v * 16, 16)
            m = akey[sl] > 0
            c = plsc.cumsum(m.astype(jnp.int32))
            plsc.store_scatter(selv, [offv + c - 1], v * 16 + iota, mask=m)
            return offv + plsc.all_reduce_population_count(m)

        lax.fori_loop(0, _NV, comp_body, jnp.zeros((16,), jnp.int32))
        first = selv[pl.ds(0, 16)][0]

        pltpu.sync_copy(bits_hbm, bkey)

        @pl.loop(0, _KV)
        def _fill_gather(v):
            sl = pl.ds(v * 16, 16)
            lanes = v * 16 + iota
            sv = jnp.where(lanes < count, selv[sl], first)
            selv[sl] = sv
            se = plsc.load_gather(sev, [sv])
            cs = se & jnp.int32(_P - 1)
            ost[sl] = cs
            oen[sl] = cs + lax.shift_right_logical(se, 13)
            osc[sl] = plsc.load_gather(bkey, [sv])

        pltpu.sync_copy(selv, sel_hbm)
        pltpu.sync_copy(ost, outs_hbm)
        pltpu.sync_copy(oen, oute_hbm)
        pltpu.sync_copy(osc, outsc_hbm)


_cp = pltpu.CompilerParams()
if "needs_layout_passes" in pltpu.CompilerParams.__dataclass_fields__:
    _cp = dataclasses.replace(_cp, needs_layout_passes=False)

_decode = functools.partial(
    pl.kernel,
    compiler_params=_cp,
    out_type=(
        jax.ShapeDtypeStruct((_K,), jnp.int32),
        jax.ShapeDtypeStruct((_K,), jnp.int32),
        jax.ShapeDtypeStruct((_K,), jnp.int32),
        jax.ShapeDtypeStruct((_K,), jnp.int32),
    ),
    mesh=plsc.VectorSubcoreMesh(core_axis_name="c", subcore_axis_name="s"),
    scratch_types=[
        pltpu.VMEM((_N,), jnp.int32),
        pltpu.VMEM((_N,), jnp.int32),
        pltpu.VMEM((_N,), jnp.int32),
        pltpu.VMEM((_N + 16,), jnp.int32),
        pltpu.VMEM((_N + 16,), jnp.int32),
        pltpu.VMEM((_BINS,), jnp.int32),
        pltpu.VMEM((_PPAD,), jnp.int32),
        pltpu.VMEM((_K,), jnp.int32),
        pltpu.VMEM((_K,), jnp.int32),
        pltpu.VMEM((_K,), jnp.int32),
        pltpu.VMEM((_K,), jnp.int32),
        pltpu.VMEM((16,), jnp.int32),
        pltpu.VMEM((32,), jnp.int32),
        pltpu.VMEM((32,), jnp.int32),
    ],
)(_sc_body)


def kernel(candidate_starts, candidate_ends, candidate_mention_scores,
           num_top_spans):
    bits = lax.bitcast_convert_type(candidate_mention_scores, jnp.int32)
    nts = jnp.broadcast_to(
        jnp.asarray(num_top_spans, jnp.int32).reshape(()), (16,))
    sel, ts, te, tb = _decode(candidate_starts, candidate_ends, bits, nts)
    return sel, ts, te, lax.bitcast_convert_type(tb, jnp.float32)
